# Initial kernel scaffold; baseline (speedup 1.0000x reference)
import jax
import jax.numpy as jnp
from jax.experimental import pallas as pl


def kernel(x, Wg, W1, W2, W3, W1s, W2s, W3s):
    raise NotImplementedError("write your pallas kernel here")


# speedup vs baseline: 1.1613x; 1.1613x over previous
import functools

import jax
import jax.numpy as jnp
from jax.experimental import pallas as pl
from jax.experimental.pallas import tpu as pltpu


def _moe_body(x_ref, wg_ref, w1_ref, w3_ref, w2_ref, o_ref, *, n_exp):
    e = pl.program_id(1)
    xb = x_ref[...]

    scores = jax.lax.dot_general(
        xb.astype(jnp.bfloat16), wg_ref[...].astype(jnp.bfloat16),
        (((1,), (1,)), ((), ())),
        preferred_element_type=jnp.float32)
    smax = jnp.max(scores, axis=-1, keepdims=True)
    ex = jnp.exp(scores - smax)
    probs = ex / jnp.sum(ex, axis=-1, keepdims=True)
    blk, E = probs.shape
    idx8 = jax.lax.broadcasted_iota(jnp.int32, (blk, E), 1)
    m1 = jnp.max(probs, axis=-1, keepdims=True)
    i1 = jnp.min(jnp.where(probs == m1, idx8, E), axis=-1, keepdims=True)
    probs2 = jnp.where(idx8 == i1, -jnp.inf, probs)
    m2 = jnp.max(probs2, axis=-1, keepdims=True)
    i2 = jnp.min(jnp.where(probs2 == m2, idx8, E), axis=-1, keepdims=True)
    wsum = m1 + m2 + 1e-9
    we = jnp.where(e == n_exp,
                   jnp.ones_like(m1),
                   jnp.where(i1 == e, m1 / wsum,
                             jnp.where(i2 == e, m2 / wsum,
                                       jnp.zeros_like(m1))))

    xbb = xb.astype(jnp.bfloat16)
    h1 = jax.lax.dot_general(xbb, w1_ref[0], (((1,), (1,)), ((), ())),
                             preferred_element_type=jnp.float32)
    h3 = jax.lax.dot_general(xbb, w3_ref[0], (((1,), (1,)), ((), ())),
                             preferred_element_type=jnp.float32)
    h = (h1 * jax.nn.sigmoid(h1) * h3).astype(jnp.bfloat16)
    yb = jax.lax.dot_general(h, w2_ref[0], (((1,), (1,)), ((), ())),
                             preferred_element_type=jnp.float32)

    @pl.when(e == 0)
    def _():
        o_ref[...] = jnp.zeros_like(o_ref)

    o_ref[...] += yb * we


def _moe_dense(xf, Wg, W1c, W3c, W2c, blk, interpret=False):
    n, dim = xf.shape
    ne, hid, _ = W1c.shape
    nb = n // blk
    E = ne - 1
    return pl.pallas_call(
        functools.partial(_moe_body, n_exp=E),
        grid=(nb, ne),
        in_specs=[
            pl.BlockSpec((blk, dim), lambda i, e: (i, 0)),
            pl.BlockSpec((E, dim), lambda i, e: (0, 0)),
            pl.BlockSpec((1, hid, dim), lambda i, e: (e, 0, 0)),
            pl.BlockSpec((1, hid, dim), lambda i, e: (e, 0, 0)),
            pl.BlockSpec((1, dim, hid), lambda i, e: (e, 0, 0)),
        ],
        out_specs=pl.BlockSpec((blk, dim), lambda i, e: (i, 0)),
        out_shape=jax.ShapeDtypeStruct((n, dim), jnp.float32),
        compiler_params=pltpu.CompilerParams(
            dimension_semantics=("parallel", "arbitrary")),
        interpret=interpret,
    )(xf, Wg, W1c, W3c, W2c)


def kernel(x, Wg, W1, W2, W3, W1s, W2s, W3s):
    bsz, seqlen, dim = x.shape
    xf = x.reshape(-1, dim)
    W1c = jnp.concatenate([W1, W1s[None]], 0).astype(jnp.bfloat16)
    W3c = jnp.concatenate([W3, W3s[None]], 0).astype(jnp.bfloat16)
    W2c = jnp.concatenate([W2, W2s[None]], 0).astype(jnp.bfloat16)
    out = _moe_dense(xf, Wg, W1c, W3c, W2c, blk=512)
    return out.reshape(bsz, seqlen, dim)


# speedup vs baseline: 1.8628x; 1.6040x over previous
import functools

import jax
import jax.numpy as jnp
from jax import lax
from jax.experimental import pallas as pl
from jax.experimental.pallas import tpu as pltpu
from jax.experimental.pallas import tpu_sc as plsc

N = 8192
DIM = 2048
HID = 1536
E = 8
BLKR = 512
BLKG = 256
NPAD = 18432
NPADT = NPAD + N
NBG1 = NPAD // BLKG
NBT = NPADT // BLKG
NW = 32
TPW = N // NW
CH = 32


def _router_body(x_ref, wg_ref, tri_ref, i1_ref, i2_ref, r1_ref,
                 r2_ref, w1_ref, w2_ref, cnt_ref, run_ref):
    pid = pl.program_id(0)
    xb = x_ref[...]
    xbf = xb.astype(jnp.bfloat16)

    scores = lax.dot_general(
        xbf, wg_ref[...].astype(jnp.bfloat16), (((1,), (1,)), ((), ())),
        preferred_element_type=jnp.float32)
    smax = jnp.max(scores, axis=-1, keepdims=True)
    ex = jnp.exp(scores - smax)
    probs = ex / jnp.sum(ex, axis=-1, keepdims=True)
    idx8 = lax.broadcasted_iota(jnp.int32, (BLKR, E), 1)
    m1 = jnp.max(probs, axis=-1, keepdims=True)
    i1 = jnp.min(jnp.where(probs == m1, idx8, E), axis=-1, keepdims=True)
    probs2 = jnp.where(idx8 == i1, -jnp.inf, probs)
    m2 = jnp.max(probs2, axis=-1, keepdims=True)
    i2 = jnp.min(jnp.where(probs2 == m2, idx8, E), axis=-1, keepdims=True)
    wsum = m1 + m2 + 1e-9
    i1_ref[...] = i1
    i2_ref[...] = i2
    w1_ref[...] = m1 / wsum
    w2_ref[...] = m2 / wsum

    oh1 = (idx8 == i1).astype(jnp.int8)
    oh2 = (idx8 == i2).astype(jnp.int8)
    oh = jnp.concatenate([oh1, oh2], axis=0)
    excl = lax.dot_general(tri_ref[...], oh, (((1,), (0,)), ((), ())),
                           preferred_element_type=jnp.int32)

    @pl.when(pid == 0)
    def _():
        run_ref[...] = jnp.zeros_like(run_ref)

    run = run_ref[...]
    tb = excl + run
    r1_ref[...] = jnp.sum(tb[:BLKR] * oh1.astype(jnp.int32), axis=-1,
                          keepdims=True)
    r2_ref[...] = jnp.sum(tb[BLKR:] * oh2.astype(jnp.int32), axis=-1,
                          keepdims=True)
    newrun = run + jnp.sum(oh.astype(jnp.int32), axis=0, keepdims=True)
    run_ref[...] = newrun
    cnt_ref[...] = newrun


def _router(xf, Wg, tri, interpret=False):
    nb = N // BLKR
    outs = pl.pallas_call(
        _router_body,
        grid=(nb,),
        in_specs=[
            pl.BlockSpec((BLKR, DIM), lambda i: (i, 0)),
            pl.BlockSpec((E, DIM), lambda i: (0, 0)),
            pl.BlockSpec((2 * BLKR, 2 * BLKR), lambda i: (0, 0)),
        ],
        out_specs=[
            pl.BlockSpec((BLKR, 1), lambda i: (i, 0)),
            pl.BlockSpec((BLKR, 1), lambda i: (i, 0)),
            pl.BlockSpec((BLKR, 1), lambda i: (i, 0)),
            pl.BlockSpec((BLKR, 1), lambda i: (i, 0)),
            pl.BlockSpec((BLKR, 1), lambda i: (i, 0)),
            pl.BlockSpec((BLKR, 1), lambda i: (i, 0)),
            pl.BlockSpec((1, E), lambda i: (0, 0)),
        ],
        out_shape=[
            jax.ShapeDtypeStruct((N, 1), jnp.int32),
            jax.ShapeDtypeStruct((N, 1), jnp.int32),
            jax.ShapeDtypeStruct((N, 1), jnp.int32),
            jax.ShapeDtypeStruct((N, 1), jnp.int32),
            jax.ShapeDtypeStruct((N, 1), jnp.float32),
            jax.ShapeDtypeStruct((N, 1), jnp.float32),
            jax.ShapeDtypeStruct((1, E), jnp.int32),
        ],
        scratch_shapes=[pltpu.VMEM((1, E), jnp.int32)],
        compiler_params=pltpu.CompilerParams(
            dimension_semantics=("arbitrary",)),
        interpret=interpret,
    )(xf, Wg, tri)
    return outs


def _dest_body(i1_ref, i2_ref, r1_ref, r2_ref, pb_ref, d0_ref, d1_ref):
    blk = i1_ref.shape[0]
    idx16 = lax.broadcasted_iota(jnp.int32, (blk, 16), 1)
    pb = pb_ref[...]
    d0_ref[...] = jnp.sum(jnp.where(i1_ref[...] == idx16, pb, 0), axis=-1,
                          keepdims=True) + r1_ref[...]
    d1_ref[...] = jnp.sum(jnp.where(i2_ref[...] == idx16, pb, 0), axis=-1,
                          keepdims=True) + r2_ref[...]


def _dest(i1, i2, r1, r2, padbase, interpret=False):
    blk = min(1024, N)
    nb = N // blk
    return pl.pallas_call(
        _dest_body,
        grid=(nb,),
        in_specs=[
            pl.BlockSpec((blk, 1), lambda i: (i, 0)),
            pl.BlockSpec((blk, 1), lambda i: (i, 0)),
            pl.BlockSpec((blk, 1), lambda i: (i, 0)),
            pl.BlockSpec((blk, 1), lambda i: (i, 0)),
            pl.BlockSpec((1, 16), lambda i: (0, 0)),
        ],
        out_specs=[
            pl.BlockSpec((blk, 1), lambda i: (i, 0)),
            pl.BlockSpec((blk, 1), lambda i: (i, 0)),
        ],
        out_shape=[
            jax.ShapeDtypeStruct((N, 1), jnp.int32),
            jax.ShapeDtypeStruct((N, 1), jnp.int32),
        ],
        compiler_params=pltpu.CompilerParams(
            dimension_semantics=("parallel",)),
        interpret=interpret,
    )(i1, i2, r1, r2, padbase)


def _sc_dispatch_body(xfr, d0r, d1r, w1r, w2r,
                      xg, wslot,
                      rowbuf, d32a, d32b, w32a, w32b, onesv):
    wid = lax.axis_index("s") * 2 + lax.axis_index("c")
    tb = wid * TPW

    ones16 = jnp.full((16,), 1.0, jnp.float32)
    for j in range(TPW // 16):
        onesv[pl.ds(16 * j, 16)] = ones16
    pltpu.sync_copy(onesv, wslot.at[pl.ds(NPAD + tb, TPW)])

    for c in range(TPW // CH):
        o = c * CH
        pltpu.sync_copy(d0r.at[pl.ds(tb + o, CH)], d32a)
        pltpu.sync_copy(d1r.at[pl.ds(tb + o, CH)], d32b)
        pltpu.sync_copy(xfr.at[pl.ds(tb + o, CH)], rowbuf)
        pltpu.sync_copy(rowbuf, xg.at[d32a])
        pltpu.sync_copy(rowbuf, xg.at[d32b])
        pltpu.sync_copy(rowbuf, xg.at[pl.ds(NPAD + tb + o, CH)])
        pltpu.sync_copy(w1r.at[pl.ds(tb + o, CH)], w32a)
        pltpu.sync_copy(w32a, wslot.at[d32a])
        pltpu.sync_copy(w2r.at[pl.ds(tb + o, CH)], w32b)
        pltpu.sync_copy(w32b, wslot.at[d32b])


def _sc_dispatch(xf, d0, d1, w1, w2):
    mesh = plsc.VectorSubcoreMesh(core_axis_name="c", subcore_axis_name="s")
    fn = pl.kernel(
        _sc_dispatch_body,
        out_type=[
            jax.ShapeDtypeStruct((NPADT, DIM), jnp.float32),
            jax.ShapeDtypeStruct((NPADT,), jnp.float32),
        ],
        mesh=mesh,
        scratch_types=[
            pltpu.VMEM((CH, DIM), jnp.float32),
            pltpu.VMEM((CH,), jnp.int32),
            pltpu.VMEM((CH,), jnp.int32),
            pltpu.VMEM((CH,), jnp.float32),
            pltpu.VMEM((CH,), jnp.float32),
            pltpu.VMEM((TPW,), jnp.float32),
        ],
    )
    return fn(xf, d0, d1, w1, w2)


def _ffn_body(be_ref, xg_ref, w1_ref, w3_ref, w2_ref, ws_ref, o_ref):
    xb = xg_ref[...].astype(jnp.bfloat16)
    h1 = lax.dot_general(xb, w1_ref[0], (((1,), (1,)), ((), ())),
                         preferred_element_type=jnp.float32)
    h3 = lax.dot_general(xb, w3_ref[0], (((1,), (1,)), ((), ())),
                         preferred_element_type=jnp.float32)
    h = (h1 * jax.nn.sigmoid(h1) * h3).astype(jnp.bfloat16)
    y = lax.dot_general(h, w2_ref[0], (((1,), (1,)), ((), ())),
                        preferred_element_type=jnp.float32)
    o_ref[...] = y * ws_ref[...]


def _ffn(block_expert, xg, W1c, W3c, W2c, wslot2d, interpret=False):
    grid_spec = pltpu.PrefetchScalarGridSpec(
        num_scalar_prefetch=1,
        grid=(NBT,),
        in_specs=[
            pl.BlockSpec((BLKG, DIM), lambda j, be: (j, 0)),
            pl.BlockSpec((1, HID, DIM), lambda j, be: (be[j], 0, 0)),
            pl.BlockSpec((1, HID, DIM), lambda j, be: (be[j], 0, 0)),
            pl.BlockSpec((1, DIM, HID), lambda j, be: (be[j], 0, 0)),
            pl.BlockSpec((BLKG, 1), lambda j, be: (j, 0)),
        ],
        out_specs=pl.BlockSpec((BLKG, DIM), lambda j, be: (j, 0)),
    )
    return pl.pallas_call(
        _ffn_body,
        grid_spec=grid_spec,
        out_shape=jax.ShapeDtypeStruct((NPADT, DIM), jnp.float32),
        compiler_params=pltpu.CompilerParams(
            dimension_semantics=("arbitrary",)),
        interpret=interpret,
    )(block_expert, xg, W1c, W3c, W2c, wslot2d)


def _sc_gather_body(outg, dest0, dest1, g0, g1,
                    rowbuf, d32):
    wid = lax.axis_index("s") * 2 + lax.axis_index("c")
    tb = wid * TPW
    for c in range(TPW // CH):
        o = c * CH
        pltpu.sync_copy(dest0.at[pl.ds(tb + o, CH)], d32)
        pltpu.sync_copy(outg.at[d32], rowbuf)
        pltpu.sync_copy(rowbuf, g0.at[pl.ds(tb + o, CH)])
        pltpu.sync_copy(dest1.at[pl.ds(tb + o, CH)], d32)
        pltpu.sync_copy(outg.at[d32], rowbuf)
        pltpu.sync_copy(rowbuf, g1.at[pl.ds(tb + o, CH)])


def _sc_gather(outg, dest0, dest1):
    mesh = plsc.VectorSubcoreMesh(core_axis_name="c", subcore_axis_name="s")
    fn = pl.kernel(
        _sc_gather_body,
        out_type=[
            jax.ShapeDtypeStruct((N, DIM), jnp.float32),
            jax.ShapeDtypeStruct((N, DIM), jnp.float32),
        ],
        mesh=mesh,
        scratch_types=[
            pltpu.VMEM((CH, DIM), jnp.float32),
            pltpu.VMEM((CH,), jnp.int32),
        ],
    )
    return fn(outg, dest0, dest1)


def _combine_body(g0_ref, g1_ref, sh_ref, o_ref):
    o_ref[...] = g0_ref[...] + g1_ref[...] + sh_ref[...]


def _combine(g0, g1, outg, interpret=False):
    blk = min(512, N)
    nb = N // blk
    base = NPAD // blk
    return pl.pallas_call(
        _combine_body,
        grid=(nb,),
        in_specs=[
            pl.BlockSpec((blk, DIM), lambda i: (i, 0)),
            pl.BlockSpec((blk, DIM), lambda i: (i, 0)),
            pl.BlockSpec((blk, DIM), lambda i: (base + i, 0)),
        ],
        out_specs=pl.BlockSpec((blk, DIM), lambda i: (i, 0)),
        out_shape=jax.ShapeDtypeStruct((N, DIM), jnp.float32),
        compiler_params=pltpu.CompilerParams(
            dimension_semantics=("parallel",)),
        interpret=interpret,
    )(g0, g1, outg)


def _metadata(counts):
    cnt_pad = ((counts + BLKG - 1) // BLKG) * BLKG
    padbase = jnp.concatenate(
        [jnp.zeros((1,), jnp.int32), jnp.cumsum(cnt_pad)[:-1].astype(jnp.int32),
         jnp.zeros((16 - E,), jnp.int32)])
    off = jnp.arange(NBG1, dtype=jnp.int32) * BLKG
    be1 = (jnp.sum((padbase[None, :E] <= off[:, None]).astype(jnp.int32),
                   axis=1) - 1).astype(jnp.int32)
    block_expert = jnp.concatenate(
        [be1, jnp.full((NBT - NBG1,), E, jnp.int32)])
    return padbase, block_expert


def kernel(x, Wg, W1, W2, W3, W1s, W2s, W3s):
    bsz, seqlen, dim = x.shape
    xf = x.reshape(-1, dim)
    ar = jnp.arange(2 * BLKR, dtype=jnp.int32)
    tri = (ar[:, None] > ar[None, :]).astype(jnp.int8)
    W1c = jnp.concatenate([W1, W1s[None]], 0).astype(jnp.bfloat16)
    W3c = jnp.concatenate([W3, W3s[None]], 0).astype(jnp.bfloat16)
    W2c = jnp.concatenate([W2, W2s[None]], 0).astype(jnp.bfloat16)

    i1, i2, r1, r2, w1, w2, cnt = _router(xf, Wg, tri)
    counts = cnt[0]
    padbase, block_expert = _metadata(counts)

    d0, d1 = _dest(i1, i2, r1, r2, padbase.reshape(1, 16))
    dest0 = d0.reshape(N)
    dest1 = d1.reshape(N)
    xg, wslot = _sc_dispatch(xf, dest0, dest1, w1.reshape(N), w2.reshape(N))

    outg = _ffn(block_expert, xg, W1c, W3c, W2c, wslot.reshape(NPADT, 1))

    g0, g1 = _sc_gather(outg, dest0, dest1)
    y = _combine(g0, g1, outg)
    return y.reshape(bsz, seqlen, dim)


# speedup vs baseline: 1.8654x; 1.0014x over previous
import functools

import jax
import jax.numpy as jnp
from jax import lax
from jax.experimental import pallas as pl
from jax.experimental.pallas import tpu as pltpu
from jax.experimental.pallas import tpu_sc as plsc

N = 8192
DIM = 2048
HID = 1536
E = 8
BLKR = 512
BLKG = 256
NPAD = 18432
NPADT = NPAD + N
NBG1 = NPAD // BLKG
NBT = NPADT // BLKG
NW = 32
TPW = N // NW
CH = 16


def _router_body(x_ref, wg_ref, tri_ref, i1_ref, i2_ref, r1_ref,
                 r2_ref, w1_ref, w2_ref, cnt_ref, run_ref):
    pid = pl.program_id(0)
    xb = x_ref[...]
    xbf = xb.astype(jnp.bfloat16)

    scores = lax.dot_general(
        xbf, wg_ref[...].astype(jnp.bfloat16), (((1,), (1,)), ((), ())),
        preferred_element_type=jnp.float32)
    smax = jnp.max(scores, axis=-1, keepdims=True)
    ex = jnp.exp(scores - smax)
    probs = ex / jnp.sum(ex, axis=-1, keepdims=True)
    idx8 = lax.broadcasted_iota(jnp.int32, (BLKR, E), 1)
    m1 = jnp.max(probs, axis=-1, keepdims=True)
    i1 = jnp.min(jnp.where(probs == m1, idx8, E), axis=-1, keepdims=True)
    probs2 = jnp.where(idx8 == i1, -jnp.inf, probs)
    m2 = jnp.max(probs2, axis=-1, keepdims=True)
    i2 = jnp.min(jnp.where(probs2 == m2, idx8, E), axis=-1, keepdims=True)
    wsum = m1 + m2 + 1e-9
    i1_ref[...] = i1
    i2_ref[...] = i2
    w1_ref[...] = m1 / wsum
    w2_ref[...] = m2 / wsum

    oh1 = (idx8 == i1).astype(jnp.int8)
    oh2 = (idx8 == i2).astype(jnp.int8)
    oh = jnp.concatenate([oh1, oh2], axis=0)
    excl = lax.dot_general(tri_ref[...], oh, (((1,), (0,)), ((), ())),
                           preferred_element_type=jnp.int32)

    @pl.when(pid == 0)
    def _():
        run_ref[...] = jnp.zeros_like(run_ref)

    run = run_ref[...]
    tb = excl + run
    r1_ref[...] = jnp.sum(tb[:BLKR] * oh1.astype(jnp.int32), axis=-1,
                          keepdims=True)
    r2_ref[...] = jnp.sum(tb[BLKR:] * oh2.astype(jnp.int32), axis=-1,
                          keepdims=True)
    newrun = run + jnp.sum(oh.astype(jnp.int32), axis=0, keepdims=True)
    run_ref[...] = newrun
    cnt_ref[...] = newrun


def _router(xf, Wg, tri, interpret=False):
    nb = N // BLKR
    outs = pl.pallas_call(
        _router_body,
        grid=(nb,),
        in_specs=[
            pl.BlockSpec((BLKR, DIM), lambda i: (i, 0)),
            pl.BlockSpec((E, DIM), lambda i: (0, 0)),
            pl.BlockSpec((2 * BLKR, 2 * BLKR), lambda i: (0, 0)),
        ],
        out_specs=[
            pl.BlockSpec((BLKR, 1), lambda i: (i, 0)),
            pl.BlockSpec((BLKR, 1), lambda i: (i, 0)),
            pl.BlockSpec((BLKR, 1), lambda i: (i, 0)),
            pl.BlockSpec((BLKR, 1), lambda i: (i, 0)),
            pl.BlockSpec((BLKR, 1), lambda i: (i, 0)),
            pl.BlockSpec((BLKR, 1), lambda i: (i, 0)),
            pl.BlockSpec((1, E), lambda i: (0, 0)),
        ],
        out_shape=[
            jax.ShapeDtypeStruct((N, 1), jnp.int32),
            jax.ShapeDtypeStruct((N, 1), jnp.int32),
            jax.ShapeDtypeStruct((N, 1), jnp.int32),
            jax.ShapeDtypeStruct((N, 1), jnp.int32),
            jax.ShapeDtypeStruct((N, 1), jnp.float32),
            jax.ShapeDtypeStruct((N, 1), jnp.float32),
            jax.ShapeDtypeStruct((1, E), jnp.int32),
        ],
        scratch_shapes=[pltpu.VMEM((1, E), jnp.int32)],
        compiler_params=pltpu.CompilerParams(
            dimension_semantics=("arbitrary",)),
        interpret=interpret,
    )(xf, Wg, tri)
    return outs


def _dest_body(i1_ref, i2_ref, r1_ref, r2_ref, pb_ref, d0_ref, d1_ref):
    blk = i1_ref.shape[0]
    idx16 = lax.broadcasted_iota(jnp.int32, (blk, 16), 1)
    pb = pb_ref[...]
    d0_ref[...] = jnp.sum(jnp.where(i1_ref[...] == idx16, pb, 0), axis=-1,
                          keepdims=True) + r1_ref[...]
    d1_ref[...] = jnp.sum(jnp.where(i2_ref[...] == idx16, pb, 0), axis=-1,
                          keepdims=True) + r2_ref[...]


def _dest(i1, i2, r1, r2, padbase, interpret=False):
    blk = min(1024, N)
    nb = N // blk
    return pl.pallas_call(
        _dest_body,
        grid=(nb,),
        in_specs=[
            pl.BlockSpec((blk, 1), lambda i: (i, 0)),
            pl.BlockSpec((blk, 1), lambda i: (i, 0)),
            pl.BlockSpec((blk, 1), lambda i: (i, 0)),
            pl.BlockSpec((blk, 1), lambda i: (i, 0)),
            pl.BlockSpec((1, 16), lambda i: (0, 0)),
        ],
        out_specs=[
            pl.BlockSpec((blk, 1), lambda i: (i, 0)),
            pl.BlockSpec((blk, 1), lambda i: (i, 0)),
        ],
        out_shape=[
            jax.ShapeDtypeStruct((N, 1), jnp.int32),
            jax.ShapeDtypeStruct((N, 1), jnp.int32),
        ],
        compiler_params=pltpu.CompilerParams(
            dimension_semantics=("parallel",)),
        interpret=interpret,
    )(i1, i2, r1, r2, padbase)


def _sc_dispatch_body(xfr, d0r, d1r, w1r, w2r, xg, wslot,
                      rb0, rb1, da0, da1, db0, db1, wa, wb,
                      sl0, sl1, sa0, sa1, sb0, sb1, swa0, swa1, swb0, swb1):
    wid = lax.axis_index("s") * 2 + lax.axis_index("c")
    tb = wid * TPW
    pltpu.sync_copy(w1r.at[pl.ds(tb, TPW)], wa)
    pltpu.sync_copy(w2r.at[pl.ds(tb, TPW)], wb)
    rb = (rb0, rb1)
    da = (da0, da1)
    db = (db0, db1)
    lsem = (sl0, sl1)
    asem = (sa0, sa1)
    bsem = (sb0, sb1)
    wasem = (swa0, swa1)
    wbsem = (swb0, swb1)
    nch = TPW // CH
    hl = [None, None]
    hA = [None, None]
    hB = [None, None]
    hWa = [None, None]
    hWb = [None, None]

    hl[0] = pltpu.async_copy(xfr.at[pl.ds(tb, CH)], rb[0], lsem[0])
    for c in range(nch):
        p = c & 1
        o = c * CH
        hl[p].wait()
        pltpu.sync_copy(d0r.at[pl.ds(tb + o, CH)], da[p])
        pltpu.sync_copy(d1r.at[pl.ds(tb + o, CH)], db[p])
        hA[p] = pltpu.async_copy(rb[p], xg.at[da[p]], asem[p])
        hB[p] = pltpu.async_copy(rb[p], xg.at[db[p]], bsem[p])
        hWa[p] = pltpu.async_copy(wa.at[pl.ds(o, CH)], wslot.at[da[p]],
                                  wasem[p])
        hWb[p] = pltpu.async_copy(wb.at[pl.ds(o, CH)], wslot.at[db[p]],
                                  wbsem[p])
        if c + 1 < nch:
            q = 1 - p
            if c >= 1:
                hA[q].wait()
                hB[q].wait()
                hWa[q].wait()
                hWb[q].wait()
            hl[q] = pltpu.async_copy(xfr.at[pl.ds(tb + o + CH, CH)], rb[q],
                                     lsem[q])
    p = (nch - 1) & 1
    hA[p].wait()
    hB[p].wait()
    hWa[p].wait()
    hWb[p].wait()


def _sc_dispatch(xf, d0, d1, w1, w2):
    mesh = plsc.VectorSubcoreMesh(core_axis_name="c", subcore_axis_name="s")
    fn = pl.kernel(
        _sc_dispatch_body,
        out_type=[
            jax.ShapeDtypeStruct((NPAD, DIM), jnp.float32),
            jax.ShapeDtypeStruct((NPAD,), jnp.float32),
        ],
        mesh=mesh,
        scratch_types=[
            pltpu.VMEM((CH, DIM), jnp.float32),
            pltpu.VMEM((CH, DIM), jnp.float32),
            pltpu.VMEM((CH,), jnp.int32),
            pltpu.VMEM((CH,), jnp.int32),
            pltpu.VMEM((CH,), jnp.int32),
            pltpu.VMEM((CH,), jnp.int32),
            pltpu.VMEM((TPW,), jnp.float32),
            pltpu.VMEM((TPW,), jnp.float32),
            pltpu.SemaphoreType.DMA,
            pltpu.SemaphoreType.DMA,
            pltpu.SemaphoreType.DMA,
            pltpu.SemaphoreType.DMA,
            pltpu.SemaphoreType.DMA,
            pltpu.SemaphoreType.DMA,
            pltpu.SemaphoreType.DMA,
            pltpu.SemaphoreType.DMA,
            pltpu.SemaphoreType.DMA,
            pltpu.SemaphoreType.DMA,
        ],
    )
    return fn(xf, d0, d1, w1, w2)


def _ffn_body(be_ref, xg_ref, xf_ref, w1_ref, w3_ref, w2_ref, ws_ref, o_ref):
    j = pl.program_id(0)
    is_sh = j >= NBG1
    xb = jnp.where(is_sh, xf_ref[...], xg_ref[...]).astype(jnp.bfloat16)
    h1 = lax.dot_general(xb, w1_ref[0], (((1,), (1,)), ((), ())),
                         preferred_element_type=jnp.float32)
    h3 = lax.dot_general(xb, w3_ref[0], (((1,), (1,)), ((), ())),
                         preferred_element_type=jnp.float32)
    h = (h1 * jax.nn.sigmoid(h1) * h3).astype(jnp.bfloat16)
    y = lax.dot_general(h, w2_ref[0], (((1,), (1,)), ((), ())),
                        preferred_element_type=jnp.float32)
    ws = jnp.where(is_sh, jnp.ones_like(ws_ref[...]), ws_ref[...])
    o_ref[...] = y * ws


def _ffn(block_expert, xg, xf, W1c, W3c, W2c, wslot2d, interpret=False):
    grid_spec = pltpu.PrefetchScalarGridSpec(
        num_scalar_prefetch=1,
        grid=(NBT,),
        in_specs=[
            pl.BlockSpec((BLKG, DIM),
                         lambda j, be: (jnp.minimum(j, NBG1 - 1), 0)),
            pl.BlockSpec((BLKG, DIM),
                         lambda j, be: (jnp.maximum(j - NBG1, 0), 0)),
            pl.BlockSpec((1, HID, DIM), lambda j, be: (be[j], 0, 0)),
            pl.BlockSpec((1, HID, DIM), lambda j, be: (be[j], 0, 0)),
            pl.BlockSpec((1, DIM, HID), lambda j, be: (be[j], 0, 0)),
            pl.BlockSpec((BLKG, 1),
                         lambda j, be: (jnp.minimum(j, NBG1 - 1), 0)),
        ],
        out_specs=pl.BlockSpec((BLKG, DIM), lambda j, be: (j, 0)),
    )
    return pl.pallas_call(
        _ffn_body,
        grid_spec=grid_spec,
        out_shape=jax.ShapeDtypeStruct((NPADT, DIM), jnp.float32),
        compiler_params=pltpu.CompilerParams(
            dimension_semantics=("arbitrary",)),
        interpret=interpret,
    )(block_expert, xg, xf, W1c, W3c, W2c, wslot2d)


def _sc_gather_body(outg, dest0, dest1, g0, g1,
                    rowbuf, d32):
    wid = lax.axis_index("s") * 2 + lax.axis_index("c")
    tb = wid * TPW
    for c in range(TPW // CH):
        o = c * CH
        pltpu.sync_copy(dest0.at[pl.ds(tb + o, CH)], d32)
        pltpu.sync_copy(outg.at[d32], rowbuf)
        pltpu.sync_copy(rowbuf, g0.at[pl.ds(tb + o, CH)])
        pltpu.sync_copy(dest1.at[pl.ds(tb + o, CH)], d32)
        pltpu.sync_copy(outg.at[d32], rowbuf)
        pltpu.sync_copy(rowbuf, g1.at[pl.ds(tb + o, CH)])


def _sc_gather(outg, dest0, dest1):
    mesh = plsc.VectorSubcoreMesh(core_axis_name="c", subcore_axis_name="s")
    fn = pl.kernel(
        _sc_gather_body,
        out_type=[
            jax.ShapeDtypeStruct((N, DIM), jnp.float32),
            jax.ShapeDtypeStruct((N, DIM), jnp.float32),
        ],
        mesh=mesh,
        scratch_types=[
            pltpu.VMEM((CH, DIM), jnp.float32),
            pltpu.VMEM((CH,), jnp.int32),
        ],
    )
    return fn(outg, dest0, dest1)


def _combine_body(g0_ref, g1_ref, sh_ref, o_ref):
    o_ref[...] = g0_ref[...] + g1_ref[...] + sh_ref[...]


def _combine(g0, g1, outg, interpret=False):
    blk = min(512, N)
    nb = N // blk
    base = NPAD // blk
    return pl.pallas_call(
        _combine_body,
        grid=(nb,),
        in_specs=[
            pl.BlockSpec((blk, DIM), lambda i: (i, 0)),
            pl.BlockSpec((blk, DIM), lambda i: (i, 0)),
            pl.BlockSpec((blk, DIM), lambda i: (base + i, 0)),
        ],
        out_specs=pl.BlockSpec((blk, DIM), lambda i: (i, 0)),
        out_shape=jax.ShapeDtypeStruct((N, DIM), jnp.float32),
        compiler_params=pltpu.CompilerParams(
            dimension_semantics=("parallel",)),
        interpret=interpret,
    )(g0, g1, outg)


def _metadata(counts):
    cnt_pad = ((counts + BLKG - 1) // BLKG) * BLKG
    padbase = jnp.concatenate(
        [jnp.zeros((1,), jnp.int32), jnp.cumsum(cnt_pad)[:-1].astype(jnp.int32),
         jnp.zeros((16 - E,), jnp.int32)])
    off = jnp.arange(NBG1, dtype=jnp.int32) * BLKG
    be1 = (jnp.sum((padbase[None, :E] <= off[:, None]).astype(jnp.int32),
                   axis=1) - 1).astype(jnp.int32)
    block_expert = jnp.concatenate(
        [be1, jnp.full((NBT - NBG1,), E, jnp.int32)])
    return padbase, block_expert


def kernel(x, Wg, W1, W2, W3, W1s, W2s, W3s):
    bsz, seqlen, dim = x.shape
    xf = x.reshape(-1, dim)
    ar = jnp.arange(2 * BLKR, dtype=jnp.int32)
    tri = (ar[:, None] > ar[None, :]).astype(jnp.int8)
    W1c = jnp.concatenate([W1, W1s[None]], 0).astype(jnp.bfloat16)
    W3c = jnp.concatenate([W3, W3s[None]], 0).astype(jnp.bfloat16)
    W2c = jnp.concatenate([W2, W2s[None]], 0).astype(jnp.bfloat16)

    i1, i2, r1, r2, w1, w2, cnt = _router(xf, Wg, tri)
    counts = cnt[0]
    padbase, block_expert = _metadata(counts)

    d0, d1 = _dest(i1, i2, r1, r2, padbase.reshape(1, 16))
    dest0 = d0.reshape(N)
    dest1 = d1.reshape(N)
    xg, wslot = _sc_dispatch(xf, dest0, dest1, w1.reshape(N), w2.reshape(N))

    outg = _ffn(block_expert, xg, xf, W1c, W3c, W2c, wslot.reshape(NPAD, 1))

    g0, g1 = _sc_gather(outg, dest0, dest1)
    y = _combine(g0, g1, outg)
    return y.reshape(bsz, seqlen, dim)
